# baseline (speedup 1.0000x reference)
import functools

import jax
import jax.numpy as jnp
import numpy as np
from jax import lax
from jax.experimental import pallas as pl
from jax.experimental.pallas import tpu as pltpu
from jax.experimental.pallas import tpu_sc as plsc

N = 10000
E = 320000
D = 128
NC, NS = 2, 16
NW = NC * NS
GRP = 128
GPW = 80
EPAD = NW * GPW * GRP
ACC_ROWS = 10240
NBUF = 2
CH = 16
NCH = GPW // CH


def _sc_aggregate(table, src_g, dst_g):
    mesh = plsc.VectorSubcoreMesh(
        core_axis_name="c", subcore_axis_name="s",
        num_cores=NC, num_subcores=NS)

    @functools.partial(
        pl.kernel,
        out_type=jax.ShapeDtypeStruct((NC * N, D), jnp.float32),
        mesh=mesh,
        scratch_types=[
            pltpu.VMEM((2, CH, GRP), jnp.int32),
            pltpu.VMEM((2, CH, GRP), jnp.int32),
            pltpu.VMEM((NBUF, GRP, D), jnp.float32),
            pltpu.VMEM((32, D), jnp.float32),
            pltpu.VMEM_SHARED((ACC_ROWS, D), jnp.float32),
            pltpu.SemaphoreType.DMA,
            pltpu.SemaphoreType.DMA,
            pltpu.SemaphoreType.DMA,
            pltpu.SemaphoreType.DMA,
            pltpu.SemaphoreType.DMA,
        ],
    )
    def agg(table_hbm, src_hbm, dst_hbm, out_hbm, sidx, didx, rows2, zbuf,
            acc_sh, isem, g0, g1, s0, s1):
        gsems = (g0, g1)
        ssems = (s0, s1)
        c = lax.axis_index("c")
        s = lax.axis_index("s")
        wid = c * NS + s

        pltpu.sync_copy(src_hbm.at[wid, pl.ds(0, CH)], sidx.at[0])
        pltpu.sync_copy(dst_hbm.at[wid, pl.ds(0, CH)], didx.at[0])
        for b in range(NBUF):
            pltpu.async_copy(table_hbm.at[sidx.at[0, b]], rows2.at[b], gsems[b])
        pltpu.async_copy(src_hbm.at[wid, pl.ds(CH, CH)], sidx.at[1], isem)
        pltpu.async_copy(dst_hbm.at[wid, pl.ds(CH, CH)], didx.at[1], isem)

        zvec = jnp.zeros((16,), jnp.float32)

        def zb(i, carry):
            r = i // (D // 16)
            q = i % (D // 16)
            zbuf[r, pl.ds(q * 16, 16)] = zvec
            return carry

        lax.fori_loop(0, 32 * (D // 16), zb, 0)

        zrows = ACC_ROWS // NS

        def zcp(j, carry):
            pltpu.sync_copy(zbuf, acc_sh.at[pl.ds(s * zrows + j * 32, 32)])
            return carry

        lax.fori_loop(0, zrows // 32, zcp, 0)
        plsc.subcore_barrier()

        for k in range(NCH):
            ib = k % 2
            si = sidx.at[ib]
            di = didx.at[ib]

            def inner(o, carry, si=si, di=di):
                for b in range(NBUF):
                    gl = o * NBUF + b
                    pltpu.make_async_copy(
                        table_hbm.at[si.at[gl]], rows2.at[b], gsems[b]).wait()
                    pltpu.async_copy(
                        rows2.at[b], acc_sh.at[di.at[gl]], ssems[b],
                        add=True).wait()
                    pltpu.async_copy(
                        table_hbm.at[si.at[gl + NBUF]], rows2.at[b], gsems[b])
                return carry

            lax.fori_loop(0, CH // NBUF - 1, inner, 0)

            if k < NCH - 1:
                pltpu.make_async_copy(
                    src_hbm.at[wid, pl.ds((k + 1) * CH, CH)],
                    sidx.at[1 - ib], isem).wait()
                pltpu.make_async_copy(
                    dst_hbm.at[wid, pl.ds((k + 1) * CH, CH)],
                    didx.at[1 - ib], isem).wait()
                for b in range(NBUF):
                    gl = CH - NBUF + b
                    pltpu.make_async_copy(
                        table_hbm.at[si.at[gl]], rows2.at[b], gsems[b]).wait()
                    pltpu.async_copy(
                        rows2.at[b], acc_sh.at[di.at[gl]], ssems[b],
                        add=True).wait()
                    pltpu.async_copy(
                        table_hbm.at[sidx.at[1 - ib, b]], rows2.at[b],
                        gsems[b])
                if k < NCH - 2:
                    pltpu.async_copy(
                        src_hbm.at[wid, pl.ds((k + 2) * CH, CH)],
                        sidx.at[ib], isem)
                    pltpu.async_copy(
                        dst_hbm.at[wid, pl.ds((k + 2) * CH, CH)],
                        didx.at[ib], isem)
            else:
                for b in range(NBUF):
                    gl = CH - NBUF + b
                    pltpu.make_async_copy(
                        table_hbm.at[si.at[gl]], rows2.at[b], gsems[b]).wait()
                    pltpu.async_copy(
                        rows2.at[b], acc_sh.at[di.at[gl]], ssems[b],
                        add=True).wait()
        plsc.subcore_barrier()

        orows = 624
        base = s * orows
        pltpu.sync_copy(
            acc_sh.at[pl.ds(base, orows)],
            out_hbm.at[pl.ds(c * N + base, orows)])

        @pl.when(s == NS - 1)
        def _tail():
            t0 = NS * orows
            pltpu.sync_copy(
                acc_sh.at[pl.ds(t0, N - t0)],
                out_hbm.at[pl.ds(c * N + t0, N - t0)])

    return agg(table, src_g, dst_g)


BLK = 1000


def _tc_layer(partials, h_prev, W, b, relu_resid):
    nblk = N // BLK
    b2 = b.reshape(1, D)

    if relu_resid:
        def body(p0_ref, p1_ref, h_ref, w_ref, b_ref, o_ref):
            agg = p0_ref[...] + p1_ref[...]
            z = jnp.dot(agg, w_ref[...], preferred_element_type=jnp.float32)
            o_ref[...] = jnp.maximum(z + b_ref[...] + h_ref[...], 0.0)

        in_specs = [
            pl.BlockSpec((BLK, D), lambda i: (i, 0)),
            pl.BlockSpec((BLK, D), lambda i: (i + nblk, 0)),
            pl.BlockSpec((BLK, D), lambda i: (i, 0)),
            pl.BlockSpec((D, D), lambda i: (0, 0)),
            pl.BlockSpec((1, D), lambda i: (0, 0)),
        ]
        args = (partials, partials, h_prev, W, b2)
    else:
        def body(p0_ref, p1_ref, w_ref, b_ref, o_ref):
            agg = p0_ref[...] + p1_ref[...]
            z = jnp.dot(agg, w_ref[...], preferred_element_type=jnp.float32)
            o_ref[...] = z + b_ref[...]

        in_specs = [
            pl.BlockSpec((BLK, D), lambda i: (i, 0)),
            pl.BlockSpec((BLK, D), lambda i: (i + nblk, 0)),
            pl.BlockSpec((D, D), lambda i: (0, 0)),
            pl.BlockSpec((1, D), lambda i: (0, 0)),
        ]
        args = (partials, partials, W, b2)

    return pl.pallas_call(
        body,
        grid=(nblk,),
        in_specs=in_specs,
        out_specs=pl.BlockSpec((BLK, D), lambda i: (i, 0)),
        out_shape=jax.ShapeDtypeStruct((N, D), jnp.float32),
    )(*args)


def kernel(x, edge_index, W0, b0, W1, b1, W2, b2):
    src = edge_index[0].astype(jnp.int32)
    dst = edge_index[1].astype(jnp.int32)
    pad = EPAD - E
    pad_src = jnp.asarray(np.arange(pad, dtype=np.int32) % N)
    pad_dst = jnp.asarray(N + (np.arange(pad, dtype=np.int32) % (ACC_ROWS - N)))
    src_g = jnp.concatenate([src, pad_src]).reshape(NW, GPW, GRP)
    dst_g = jnp.concatenate([dst, pad_dst]).reshape(NW, GPW, GRP)

    p = _sc_aggregate(x, src_g, dst_g)
    h1 = _tc_layer(p, x, W0, b0, True)
    p = _sc_aggregate(h1, src_g, dst_g)
    h2 = _tc_layer(p, h1, W1, b1, True)
    p = _sc_aggregate(h2, src_g, dst_g)
    return _tc_layer(p, None, W2, b2, False)


# speedup vs baseline: 1.0490x; 1.0490x over previous
import functools

import jax
import jax.numpy as jnp
import numpy as np
from jax import lax
from jax.experimental import pallas as pl
from jax.experimental.pallas import tpu as pltpu
from jax.experimental.pallas import tpu_sc as plsc

N = 10000
E = 320000
D = 128
NC, NS = 2, 16
NW = NC * NS
GRP = 128
GPW = 80
EPAD = NW * GPW * GRP
ACC_ROWS = 10240
NBUF = 2
CH = 16
NCH = GPW // CH


def _sc_aggregate(table, edges_g):
    mesh = plsc.VectorSubcoreMesh(
        core_axis_name="c", subcore_axis_name="s",
        num_cores=NC, num_subcores=NS)

    @functools.partial(
        pl.kernel,
        out_type=jax.ShapeDtypeStruct((NC * N, D), jnp.float32),
        mesh=mesh,
        scratch_types=[
            pltpu.VMEM((2, CH, GRP), jnp.int32),
            pltpu.VMEM((2, CH, GRP), jnp.int32),
            pltpu.VMEM((NBUF, GRP, D), jnp.float32),
            pltpu.VMEM((32, D), jnp.float32),
            pltpu.VMEM_SHARED((ACC_ROWS, D), jnp.float32),
            pltpu.SemaphoreType.DMA,
            pltpu.SemaphoreType.DMA,
            pltpu.SemaphoreType.DMA,
            pltpu.SemaphoreType.DMA,
            pltpu.SemaphoreType.DMA,
        ],
    )
    def agg(table_hbm, edges_hbm, out_hbm, sidx, didx, rows2, zbuf,
            acc_sh, isem, g0, g1, s0, s1):
        gsems = (g0, g1)
        ssems = (s0, s1)
        c = lax.axis_index("c")
        s = lax.axis_index("s")
        wid = c * NS + s

        pltpu.sync_copy(edges_hbm.at[0, wid, pl.ds(0, CH)], sidx.at[0])
        pltpu.sync_copy(edges_hbm.at[1, wid, pl.ds(0, CH)], didx.at[0])
        for b in range(NBUF):
            pltpu.async_copy(table_hbm.at[sidx.at[0, b]], rows2.at[b], gsems[b])
        pltpu.async_copy(edges_hbm.at[0, wid, pl.ds(CH, CH)], sidx.at[1], isem)
        pltpu.async_copy(edges_hbm.at[1, wid, pl.ds(CH, CH)], didx.at[1], isem)

        zvec = jnp.zeros((16,), jnp.float32)

        def zb(i, carry):
            r = i // (D // 16)
            q = i % (D // 16)
            zbuf[r, pl.ds(q * 16, 16)] = zvec
            return carry

        lax.fori_loop(0, 32 * (D // 16), zb, 0)

        zrows = ACC_ROWS // NS

        def zcp(j, carry):
            pltpu.sync_copy(zbuf, acc_sh.at[pl.ds(s * zrows + j * 32, 32)])
            return carry

        lax.fori_loop(0, zrows // 32, zcp, 0)
        plsc.subcore_barrier()

        for k in range(NCH):
            ib = k % 2
            si = sidx.at[ib]
            di = didx.at[ib]

            def inner(o, carry, si=si, di=di):
                for b in range(NBUF):
                    gl = o * NBUF + b
                    pltpu.make_async_copy(
                        table_hbm.at[si.at[gl]], rows2.at[b], gsems[b]).wait()
                    pltpu.async_copy(
                        rows2.at[b], acc_sh.at[di.at[gl]], ssems[b],
                        add=True).wait()
                    pltpu.async_copy(
                        table_hbm.at[si.at[gl + NBUF]], rows2.at[b], gsems[b])
                return carry

            lax.fori_loop(0, CH // NBUF - 1, inner, 0)

            if k < NCH - 1:
                pltpu.make_async_copy(
                    edges_hbm.at[0, wid, pl.ds((k + 1) * CH, CH)],
                    sidx.at[1 - ib], isem).wait()
                pltpu.make_async_copy(
                    edges_hbm.at[1, wid, pl.ds((k + 1) * CH, CH)],
                    didx.at[1 - ib], isem).wait()
                for b in range(NBUF):
                    gl = CH - NBUF + b
                    pltpu.make_async_copy(
                        table_hbm.at[si.at[gl]], rows2.at[b], gsems[b]).wait()
                    pltpu.async_copy(
                        rows2.at[b], acc_sh.at[di.at[gl]], ssems[b],
                        add=True).wait()
                    pltpu.async_copy(
                        table_hbm.at[sidx.at[1 - ib, b]], rows2.at[b],
                        gsems[b])
                if k < NCH - 2:
                    pltpu.async_copy(
                        edges_hbm.at[0, wid, pl.ds((k + 2) * CH, CH)],
                        sidx.at[ib], isem)
                    pltpu.async_copy(
                        edges_hbm.at[1, wid, pl.ds((k + 2) * CH, CH)],
                        didx.at[ib], isem)
            else:
                for b in range(NBUF):
                    gl = CH - NBUF + b
                    pltpu.make_async_copy(
                        table_hbm.at[si.at[gl]], rows2.at[b], gsems[b]).wait()
                    pltpu.async_copy(
                        rows2.at[b], acc_sh.at[di.at[gl]], ssems[b],
                        add=True).wait()
        plsc.subcore_barrier()

        orows = 624
        base = s * orows
        pltpu.sync_copy(
            acc_sh.at[pl.ds(base, orows)],
            out_hbm.at[pl.ds(c * N + base, orows)])

        @pl.when(s == NS - 1)
        def _tail():
            t0 = NS * orows
            pltpu.sync_copy(
                acc_sh.at[pl.ds(t0, N - t0)],
                out_hbm.at[pl.ds(c * N + t0, N - t0)])

    return agg(table, edges_g)


BLK = 2000


def _tc_layer(partials, h_prev, W, b, relu_resid):
    nblk = N // BLK
    b2 = b.reshape(1, D)

    if relu_resid:
        def body(p0_ref, p1_ref, h_ref, w_ref, b_ref, o_ref):
            agg = p0_ref[...] + p1_ref[...]
            z = jnp.dot(agg, w_ref[...], preferred_element_type=jnp.float32)
            o_ref[...] = jnp.maximum(z + b_ref[...] + h_ref[...], 0.0)

        in_specs = [
            pl.BlockSpec((BLK, D), lambda i: (i, 0)),
            pl.BlockSpec((BLK, D), lambda i: (i + nblk, 0)),
            pl.BlockSpec((BLK, D), lambda i: (i, 0)),
            pl.BlockSpec((D, D), lambda i: (0, 0)),
            pl.BlockSpec((1, D), lambda i: (0, 0)),
        ]
        args = (partials, partials, h_prev, W, b2)
    else:
        def body(p0_ref, p1_ref, w_ref, b_ref, o_ref):
            agg = p0_ref[...] + p1_ref[...]
            z = jnp.dot(agg, w_ref[...], preferred_element_type=jnp.float32)
            o_ref[...] = z + b_ref[...]

        in_specs = [
            pl.BlockSpec((BLK, D), lambda i: (i, 0)),
            pl.BlockSpec((BLK, D), lambda i: (i + nblk, 0)),
            pl.BlockSpec((D, D), lambda i: (0, 0)),
            pl.BlockSpec((1, D), lambda i: (0, 0)),
        ]
        args = (partials, partials, W, b2)

    return pl.pallas_call(
        body,
        grid=(nblk,),
        in_specs=in_specs,
        out_specs=pl.BlockSpec((BLK, D), lambda i: (i, 0)),
        out_shape=jax.ShapeDtypeStruct((N, D), jnp.float32),
    )(*args)


def kernel(x, edge_index, W0, b0, W1, b1, W2, b2):
    pad = EPAD - E
    pad_pair = jnp.asarray(np.stack([
        np.arange(pad, dtype=np.int32) % N,
        N + (np.arange(pad, dtype=np.int32) % (ACC_ROWS - N)),
    ]))
    edges_g = jnp.concatenate(
        [edge_index.astype(jnp.int32), pad_pair], axis=1,
    ).reshape(2, NW, GPW, GRP)

    p = _sc_aggregate(x, edges_g)
    h1 = _tc_layer(p, x, W0, b0, True)
    p = _sc_aggregate(h1, edges_g)
    h2 = _tc_layer(p, h1, W1, b1, True)
    p = _sc_aggregate(h2, edges_g)
    return _tc_layer(p, None, W2, b2, False)
